# Initial kernel scaffold; baseline (speedup 1.0000x reference)
import jax
import jax.numpy as jnp
from jax.experimental import pallas as pl


def kernel(user_id, gender, age, occupation, movie_id, genres, rating, implicit, emb_user, emb_gender, emb_age, emb_occ, emb_movie, genre_W, genre_b, ug_W, ug_b, b1_W, b1_b, b1_g, b1_beta, b1_pW, b1_pb, b2_W, b2_b, b2_g, b2_beta, b3_W, b3_b, b3_g, b3_beta, b3_pW, b3_pb, out_W, out_b):
    raise NotImplementedError("write your pallas kernel here")


# speedup vs baseline: 1.4593x; 1.4593x over previous
import functools

import jax
import jax.numpy as jnp
from jax import lax
from jax.experimental import pallas as pl
from jax.experimental.pallas import tpu as pltpu
from jax.experimental.pallas import tpu_sc as plsc

_B = 16384
_TILE = 512
_GRID = _B // _TILE
_F = 256
_NC = 2
_NS = 16
_BPW = _B // (_NC * _NS)


def _sc_gather_pair(uid, mid, utab, mtab):
    mesh = plsc.VectorSubcoreMesh(core_axis_name="c", subcore_axis_name="s")

    def body(uid_hbm, mid_hbm, ut_hbm, mt_hbm, uo_hbm, mo_hbm, idx_v, rows_v, sem):
        wid = lax.axis_index("s") * _NC + lax.axis_index("c")
        base = wid * _BPW
        pltpu.sync_copy(uid_hbm.at[pl.ds(base, _BPW)], idx_v)
        pltpu.async_copy(ut_hbm.at[idx_v], rows_v, sem).wait()
        pltpu.sync_copy(rows_v, uo_hbm.at[pl.ds(base, _BPW)])
        pltpu.sync_copy(mid_hbm.at[pl.ds(base, _BPW)], idx_v)
        pltpu.async_copy(mt_hbm.at[idx_v], rows_v, sem).wait()
        pltpu.sync_copy(rows_v, mo_hbm.at[pl.ds(base, _BPW)])

    f = pl.kernel(
        body,
        out_type=(
            jax.ShapeDtypeStruct((_B, 64), jnp.float32),
            jax.ShapeDtypeStruct((_B, 64), jnp.float32),
        ),
        mesh=mesh,
        scratch_types=[
            pltpu.VMEM((_BPW,), jnp.int32),
            pltpu.VMEM((_BPW, 64), jnp.float32),
            pltpu.SemaphoreType.DMA,
        ],
        compiler_params=pltpu.CompilerParams(use_tc_tiling_on_sc=False),
    )
    return f(uid, mid, utab, mtab)


def _k1(u_ref, mv_ref, ge_ref, ag_ref, oc_ref, feats_ref,
        T_ref, Q_ref, P_ref, GW_ref, gb_ref, ugb_ref,
        EG_ref, EA_ref, EO_ref,
        W1_ref, pW1_ref, pb1_ref,
        y1_ref, y1p_ref, s_ref, q_ref, x_scr):
    i = pl.program_id(0)
    f32 = jnp.float32
    feats = feats_ref[...]
    gv = jnp.dot(feats, GW_ref[...], preferred_element_type=f32) + gb_ref[...]
    u = u_ref[...]
    tmp = jnp.dot(u, T_ref[...], preferred_element_type=f32)
    gvr = jnp.dot(gv, Q_ref[...], preferred_element_type=f32)
    cross = jnp.dot(tmp * gvr, P_ref[...], preferred_element_type=f32) + ugb_ref[...]
    ge_oh = (ge_ref[...] == lax.broadcasted_iota(jnp.int32, (_TILE, 8), 1)).astype(f32)
    ag_oh = (ag_ref[...] == lax.broadcasted_iota(jnp.int32, (_TILE, 8), 1)).astype(f32)
    oc_oh = (oc_ref[...] == lax.broadcasted_iota(jnp.int32, (_TILE, 32), 1)).astype(f32)
    ge_emb = jnp.dot(ge_oh, EG_ref[...], preferred_element_type=f32)
    ag_emb = jnp.dot(ag_oh, EA_ref[...], preferred_element_type=f32)
    oc_emb = jnp.dot(oc_oh, EO_ref[...], preferred_element_type=f32)

    @pl.when(i == 0)
    def _():
        x_scr[...] = jnp.zeros_like(x_scr)

    x_scr[:, 0:64] = u
    x_scr[:, 64:128] = mv_ref[...]
    x_scr[:, 128:160] = gv
    x_scr[:, 160:192] = cross
    x_scr[:, 192:208] = oc_emb
    x_scr[:, 208:216] = ge_emb
    x_scr[:, 216:224] = ag_emb
    x_scr[:, 224:226] = feats[:, 19:21]
    x = x_scr[...]
    y1 = jnp.dot(x, W1_ref[...], preferred_element_type=f32)
    y1p = jnp.dot(x, pW1_ref[...], preferred_element_type=f32) + pb1_ref[...]
    y1_ref[...] = y1
    y1p_ref[...] = y1p

    @pl.when(i == 0)
    def _():
        s_ref[...] = jnp.zeros_like(s_ref)
        q_ref[...] = jnp.zeros_like(q_ref)

    s_ref[...] += jnp.sum(y1, axis=0, keepdims=True)
    q_ref[...] += jnp.sum(y1 * y1, axis=0, keepdims=True)


def _bn_scale_shift(s_ref, q_ref, g_ref, be_ref):
    mu = s_ref[...] * (1.0 / _B)
    var = q_ref[...] * (1.0 / _B) - mu * mu
    scale = lax.rsqrt(var + 1e-5) * g_ref[...]
    shift = be_ref[...] - mu * scale
    return scale, shift


def _k2(y1_ref, y1p_ref, s_ref, q_ref, g_ref, be_ref, W2_ref,
        h1_ref, y2_ref, s2_ref, q2_ref):
    i = pl.program_id(0)
    scale, shift = _bn_scale_shift(s_ref, q_ref, g_ref, be_ref)
    h1 = jnp.maximum(y1_ref[...] * scale + shift, 0.0) + y1p_ref[...]
    h1_ref[...] = h1
    y2 = jnp.dot(h1, W2_ref[...], preferred_element_type=jnp.float32)
    y2_ref[...] = y2

    @pl.when(i == 0)
    def _():
        s2_ref[...] = jnp.zeros_like(s2_ref)
        q2_ref[...] = jnp.zeros_like(q2_ref)

    s2_ref[...] += jnp.sum(y2, axis=0, keepdims=True)
    q2_ref[...] += jnp.sum(y2 * y2, axis=0, keepdims=True)


def _k3(h1_ref, y2_ref, s_ref, q_ref, g_ref, be_ref, W3_ref, pW3_ref, pb3_ref,
        y3_ref, y3p_ref, s3_ref, q3_ref):
    i = pl.program_id(0)
    scale, shift = _bn_scale_shift(s_ref, q_ref, g_ref, be_ref)
    h2 = jnp.maximum(y2_ref[...] * scale + shift, 0.0) + h1_ref[...]
    y3 = jnp.dot(h2, W3_ref[...], preferred_element_type=jnp.float32)
    y3p = jnp.dot(h2, pW3_ref[...], preferred_element_type=jnp.float32) + pb3_ref[...]
    y3_ref[...] = y3
    y3p_ref[...] = y3p

    @pl.when(i == 0)
    def _():
        s3_ref[...] = jnp.zeros_like(s3_ref)
        q3_ref[...] = jnp.zeros_like(q3_ref)

    s3_ref[...] += jnp.sum(y3, axis=0, keepdims=True)
    q3_ref[...] += jnp.sum(y3 * y3, axis=0, keepdims=True)


def _k4(y3_ref, y3p_ref, s_ref, q_ref, g_ref, be_ref, ow_ref, ob_ref, out_ref):
    scale, shift = _bn_scale_shift(s_ref, q_ref, g_ref, be_ref)
    h3 = jnp.maximum(y3_ref[...] * scale + shift, 0.0) + y3p_ref[...]
    out_ref[...] = jnp.sum(h3 * ow_ref[...], axis=1, keepdims=True) + ob_ref[...]


def _const_spec(shape):
    nd = len(shape)
    return pl.BlockSpec(shape, lambda i: (0,) * nd)


def _tile_spec(cols):
    return pl.BlockSpec((_TILE, cols), lambda i: (i, 0))


_SEQ = pltpu.CompilerParams(dimension_semantics=("arbitrary",))


def kernel(user_id, gender, age, occupation, movie_id, genres, rating, implicit,
           emb_user, emb_gender, emb_age, emb_occ, emb_movie,
           genre_W, genre_b, ug_W, ug_b,
           b1_W, b1_b, b1_g, b1_beta, b1_pW, b1_pb,
           b2_W, b2_b, b2_g, b2_beta,
           b3_W, b3_b, b3_g, b3_beta, b3_pW, b3_pb,
           out_W, out_b):
    f32 = jnp.float32
    uid = user_id.astype(jnp.int32)
    mid = movie_id.astype(jnp.int32)
    u, mv = _sc_gather_pair(uid, mid, emb_user, emb_movie)

    ge2 = gender.astype(jnp.int32).reshape(_B, 1)
    ag2 = age.astype(jnp.int32).reshape(_B, 1)
    oc2 = occupation.astype(jnp.int32).reshape(_B, 1)
    feats = jnp.concatenate(
        [genres, rating[:, None], implicit[:, None],
         jnp.zeros((_B, 11), f32)], axis=1)

    T = ug_W.reshape(64, 1024)
    c = jnp.arange(1024)
    Q = (jnp.arange(32)[:, None] == (c[None, :] // 32)).astype(f32)
    P = ((c[:, None] % 32) == jnp.arange(32)[None, :]).astype(f32)
    GW = jnp.concatenate([genre_W, jnp.zeros((13, 32), f32)], axis=0)
    gb = genre_b.reshape(1, 32)
    ugb = ug_b.reshape(1, 32)
    EG = jnp.concatenate([emb_gender, jnp.zeros((4, 8), f32)], axis=0)

    def permute_w1(W):
        return jnp.concatenate(
            [W[0:64], W[96:160], W[160:192], W[194:226], W[80:96],
             W[64:72], W[72:80], W[192:193], W[193:194],
             jnp.zeros((30, W.shape[1]), f32)], axis=0)

    W1 = permute_w1(b1_W)
    pW1 = permute_w1(b1_pW)

    y1, y1p, s1, q1 = pl.pallas_call(
        _k1,
        grid=(_GRID,),
        in_specs=[
            _tile_spec(64), _tile_spec(64), _tile_spec(1), _tile_spec(1),
            _tile_spec(1), _tile_spec(32),
            _const_spec((64, 1024)), _const_spec((32, 1024)),
            _const_spec((1024, 32)), _const_spec((32, 32)),
            _const_spec((1, 32)), _const_spec((1, 32)),
            _const_spec((8, 8)), _const_spec((8, 8)), _const_spec((32, 16)),
            _const_spec((_F, 1024)), _const_spec((_F, 1024)),
            _const_spec((1, 1024)),
        ],
        out_specs=[
            _tile_spec(1024), _tile_spec(1024),
            _const_spec((1, 1024)), _const_spec((1, 1024)),
        ],
        out_shape=[
            jax.ShapeDtypeStruct((_B, 1024), f32),
            jax.ShapeDtypeStruct((_B, 1024), f32),
            jax.ShapeDtypeStruct((1, 1024), f32),
            jax.ShapeDtypeStruct((1, 1024), f32),
        ],
        scratch_shapes=[pltpu.VMEM((_TILE, _F), f32)],
        compiler_params=_SEQ,
    )(u, mv, ge2, ag2, oc2, feats, T, Q, P, GW, gb, ugb, EG, emb_age, emb_occ,
      W1, pW1, b1_pb.reshape(1, 1024))

    h1, y2, s2, q2 = pl.pallas_call(
        _k2,
        grid=(_GRID,),
        in_specs=[
            _tile_spec(1024), _tile_spec(1024),
            _const_spec((1, 1024)), _const_spec((1, 1024)),
            _const_spec((1, 1024)), _const_spec((1, 1024)),
            _const_spec((1024, 1024)),
        ],
        out_specs=[
            _tile_spec(1024), _tile_spec(1024),
            _const_spec((1, 1024)), _const_spec((1, 1024)),
        ],
        out_shape=[
            jax.ShapeDtypeStruct((_B, 1024), f32),
            jax.ShapeDtypeStruct((_B, 1024), f32),
            jax.ShapeDtypeStruct((1, 1024), f32),
            jax.ShapeDtypeStruct((1, 1024), f32),
        ],
        compiler_params=_SEQ,
    )(y1, y1p, s1, q1, b1_g.reshape(1, 1024), b1_beta.reshape(1, 1024), b2_W)

    y3, y3p, s3, q3 = pl.pallas_call(
        _k3,
        grid=(_GRID,),
        in_specs=[
            _tile_spec(1024), _tile_spec(1024),
            _const_spec((1, 1024)), _const_spec((1, 1024)),
            _const_spec((1, 1024)), _const_spec((1, 1024)),
            _const_spec((1024, 512)), _const_spec((1024, 512)),
            _const_spec((1, 512)),
        ],
        out_specs=[
            _tile_spec(512), _tile_spec(512),
            _const_spec((1, 512)), _const_spec((1, 512)),
        ],
        out_shape=[
            jax.ShapeDtypeStruct((_B, 512), f32),
            jax.ShapeDtypeStruct((_B, 512), f32),
            jax.ShapeDtypeStruct((1, 512), f32),
            jax.ShapeDtypeStruct((1, 512), f32),
        ],
        compiler_params=_SEQ,
    )(h1, y2, s2, q2, b2_g.reshape(1, 1024), b2_beta.reshape(1, 1024),
      b3_W, b3_pW, b3_pb.reshape(1, 512))

    out = pl.pallas_call(
        _k4,
        grid=(_GRID,),
        in_specs=[
            _tile_spec(512), _tile_spec(512),
            _const_spec((1, 512)), _const_spec((1, 512)),
            _const_spec((1, 512)), _const_spec((1, 512)),
            _const_spec((1, 512)), _const_spec((1, 1)),
        ],
        out_specs=_tile_spec(1),
        out_shape=jax.ShapeDtypeStruct((_B, 1), f32),
        compiler_params=_SEQ,
    )(y3, y3p, s3, q3, b3_g.reshape(1, 512), b3_beta.reshape(1, 512),
      out_W.reshape(1, 512), out_b.reshape(1, 1))

    return out[:, 0]


# speedup vs baseline: 1.5697x; 1.0757x over previous
import functools

import jax
import jax.numpy as jnp
from jax import lax
from jax.experimental import pallas as pl
from jax.experimental.pallas import tpu as pltpu
from jax.experimental.pallas import tpu_sc as plsc

_B = 16384
_TILE = 512
_GRID = _B // _TILE
_F = 256
_NC = 2
_NS = 16
_BPW = _B // (_NC * _NS)


def _sc_gather_pair(uid, mid, utab, mtab):
    mesh = plsc.VectorSubcoreMesh(core_axis_name="c", subcore_axis_name="s")

    def body(uid_hbm, mid_hbm, ut_hbm, mt_hbm, uo_hbm, mo_hbm, idx_v, rows_v, sem):
        wid = lax.axis_index("s") * _NC + lax.axis_index("c")
        base = wid * _BPW
        pltpu.sync_copy(uid_hbm.at[pl.ds(base, _BPW)], idx_v)
        pltpu.async_copy(ut_hbm.at[idx_v], rows_v, sem).wait()
        pltpu.sync_copy(rows_v, uo_hbm.at[pl.ds(base, _BPW)])
        pltpu.sync_copy(mid_hbm.at[pl.ds(base, _BPW)], idx_v)
        pltpu.async_copy(mt_hbm.at[idx_v], rows_v, sem).wait()
        pltpu.sync_copy(rows_v, mo_hbm.at[pl.ds(base, _BPW)])

    f = pl.kernel(
        body,
        out_type=(
            jax.ShapeDtypeStruct((_B, 64), jnp.float32),
            jax.ShapeDtypeStruct((_B, 64), jnp.float32),
        ),
        mesh=mesh,
        scratch_types=[
            pltpu.VMEM((_BPW,), jnp.int32),
            pltpu.VMEM((_BPW, 64), jnp.float32),
            pltpu.SemaphoreType.DMA,
        ],
        compiler_params=pltpu.CompilerParams(use_tc_tiling_on_sc=False),
    )
    return f(uid, mid, utab, mtab)


def _k1(u_ref, mv_ref, ge_ref, ag_ref, oc_ref, feats_ref,
        T_ref, Q_ref, P_ref, GW_ref, gb_ref, ugb_ref,
        EG_ref, EA_ref, EO_ref,
        x_ref, S_ref, m_ref, x_scr):
    i = pl.program_id(0)
    f32 = jnp.float32
    bf16 = jnp.bfloat16
    feats = feats_ref[...]
    gv = jnp.dot(feats, GW_ref[...], preferred_element_type=f32) + gb_ref[...]
    u = u_ref[...]
    tmp = jnp.dot(u.astype(bf16), T_ref[...], preferred_element_type=f32)
    gvr = jnp.dot(gv.astype(bf16), Q_ref[...], preferred_element_type=f32)
    cross = jnp.dot((tmp * gvr).astype(bf16), P_ref[...],
                    preferred_element_type=f32) + ugb_ref[...]
    ge_oh = (ge_ref[...] == lax.broadcasted_iota(jnp.int32, (_TILE, 8), 1)).astype(f32)
    ag_oh = (ag_ref[...] == lax.broadcasted_iota(jnp.int32, (_TILE, 8), 1)).astype(f32)
    oc_oh = (oc_ref[...] == lax.broadcasted_iota(jnp.int32, (_TILE, 32), 1)).astype(f32)
    ge_emb = jnp.dot(ge_oh, EG_ref[...], preferred_element_type=f32)
    ag_emb = jnp.dot(ag_oh, EA_ref[...], preferred_element_type=f32)
    oc_emb = jnp.dot(oc_oh, EO_ref[...], preferred_element_type=f32)

    @pl.when(i == 0)
    def _():
        x_scr[...] = jnp.zeros_like(x_scr)

    x_scr[:, 0:64] = u
    x_scr[:, 64:128] = mv_ref[...]
    x_scr[:, 128:160] = gv
    x_scr[:, 160:192] = cross
    x_scr[:, 192:208] = oc_emb
    x_scr[:, 208:216] = ge_emb
    x_scr[:, 216:224] = ag_emb
    x_scr[:, 224:226] = jnp.zeros((_TILE, 2), f32)
    x_ref[...] = x_scr[...].astype(bf16)
    x_scr[:, 224:226] = feats[:, 19:21]
    x = x_scr[...]

    @pl.when(i == 0)
    def _():
        S_ref[...] = jnp.zeros_like(S_ref)
        m_ref[...] = jnp.zeros_like(m_ref)

    S_ref[...] += lax.dot_general(x, x, (((0,), (0,)), ((), ())),
                                  preferred_element_type=f32)
    m_ref[...] += jnp.sum(x, axis=0, keepdims=True)


def _k2(x_ref, feats_ref, S_ref, m_ref, W1f_ref, g_ref, be_ref,
        W1_ref, pW1_ref, pb1_ref, Wr_ref, pWr_ref, W2_ref,
        h1_ref, y2_ref, s2_ref, q2_ref, sc_scr, sh_scr):
    i = pl.program_id(0)
    f32 = jnp.float32
    bf16 = jnp.bfloat16

    @pl.when(i == 0)
    def _():
        W1f = W1f_ref[...]
        SW = jnp.dot(S_ref[...], W1f, preferred_element_type=f32)
        ey2 = jnp.sum(W1f * SW, axis=0, keepdims=True) * (1.0 / _B)
        mu = jnp.dot(m_ref[...], W1f, preferred_element_type=f32) * (1.0 / _B)
        var = ey2 - mu * mu
        scale = lax.rsqrt(var + 1e-5) * g_ref[...]
        sc_scr[...] = scale
        sh_scr[...] = be_ref[...] - mu * scale
        s2_ref[...] = jnp.zeros_like(s2_ref)
        q2_ref[...] = jnp.zeros_like(q2_ref)

    x = x_ref[...]
    rt = feats_ref[:, 19:20]
    im = feats_ref[:, 20:21]
    y1 = (jnp.dot(x, W1_ref[...], preferred_element_type=f32)
          + rt * Wr_ref[0:1, :] + im * Wr_ref[1:2, :])
    y1p = (jnp.dot(x, pW1_ref[...], preferred_element_type=f32)
           + rt * pWr_ref[0:1, :] + im * pWr_ref[1:2, :] + pb1_ref[...])
    h1 = jnp.maximum(y1 * sc_scr[...] + sh_scr[...], 0.0) + y1p
    h1_ref[...] = h1.astype(bf16)
    y2 = jnp.dot(h1.astype(bf16), W2_ref[...], preferred_element_type=f32)
    y2_ref[...] = y2
    s2_ref[...] += jnp.sum(y2, axis=0, keepdims=True)
    q2_ref[...] += jnp.sum(y2 * y2, axis=0, keepdims=True)


def _bn_scale_shift(s_ref, q_ref, g_ref, be_ref):
    mu = s_ref[...] * (1.0 / _B)
    var = q_ref[...] * (1.0 / _B) - mu * mu
    scale = lax.rsqrt(var + 1e-5) * g_ref[...]
    shift = be_ref[...] - mu * scale
    return scale, shift


def _k3(h1_ref, y2_ref, s_ref, q_ref, g_ref, be_ref, W3_ref, pW3_ref, pb3_ref,
        y3_ref, y3p_ref, s3_ref, q3_ref):
    i = pl.program_id(0)
    f32 = jnp.float32
    bf16 = jnp.bfloat16
    scale, shift = _bn_scale_shift(s_ref, q_ref, g_ref, be_ref)
    h2 = jnp.maximum(y2_ref[...] * scale + shift, 0.0) + h1_ref[...].astype(f32)
    h2b = h2.astype(bf16)
    y3 = jnp.dot(h2b, W3_ref[...], preferred_element_type=f32)
    y3p = jnp.dot(h2b, pW3_ref[...], preferred_element_type=f32) + pb3_ref[...]
    y3_ref[...] = y3
    y3p_ref[...] = y3p

    @pl.when(i == 0)
    def _():
        s3_ref[...] = jnp.zeros_like(s3_ref)
        q3_ref[...] = jnp.zeros_like(q3_ref)

    s3_ref[...] += jnp.sum(y3, axis=0, keepdims=True)
    q3_ref[...] += jnp.sum(y3 * y3, axis=0, keepdims=True)


def _k4(y3_ref, y3p_ref, s_ref, q_ref, g_ref, be_ref, ow_ref, ob_ref, out_ref):
    scale, shift = _bn_scale_shift(s_ref, q_ref, g_ref, be_ref)
    h3 = jnp.maximum(y3_ref[...] * scale + shift, 0.0) + y3p_ref[...]
    out_ref[...] = jnp.sum(h3 * ow_ref[...], axis=1, keepdims=True) + ob_ref[...]


def _const_spec(shape):
    nd = len(shape)
    return pl.BlockSpec(shape, lambda i: (0,) * nd)


def _tile_spec(cols):
    return pl.BlockSpec((_TILE, cols), lambda i: (i, 0))


_SEQ = pltpu.CompilerParams(dimension_semantics=("arbitrary",))


def kernel(user_id, gender, age, occupation, movie_id, genres, rating, implicit,
           emb_user, emb_gender, emb_age, emb_occ, emb_movie,
           genre_W, genre_b, ug_W, ug_b,
           b1_W, b1_b, b1_g, b1_beta, b1_pW, b1_pb,
           b2_W, b2_b, b2_g, b2_beta,
           b3_W, b3_b, b3_g, b3_beta, b3_pW, b3_pb,
           out_W, out_b):
    f32 = jnp.float32
    bf16 = jnp.bfloat16
    uid = user_id.astype(jnp.int32)
    mid = movie_id.astype(jnp.int32)
    u, mv = _sc_gather_pair(uid, mid, emb_user, emb_movie)

    ge2 = gender.astype(jnp.int32).reshape(_B, 1)
    ag2 = age.astype(jnp.int32).reshape(_B, 1)
    oc2 = occupation.astype(jnp.int32).reshape(_B, 1)
    feats = jnp.concatenate(
        [genres, rating[:, None], implicit[:, None],
         jnp.zeros((_B, 11), f32)], axis=1)

    T = ug_W.reshape(64, 1024).astype(bf16)
    c = jnp.arange(1024)
    Q = (jnp.arange(32)[:, None] == (c[None, :] // 32)).astype(bf16)
    P = ((c[:, None] % 32) == jnp.arange(32)[None, :]).astype(bf16)
    GW = jnp.concatenate([genre_W, jnp.zeros((13, 32), f32)], axis=0)
    gb = genre_b.reshape(1, 32)
    ugb = ug_b.reshape(1, 32)
    EG = jnp.concatenate([emb_gender, jnp.zeros((4, 8), f32)], axis=0)

    def permute_w1(W):
        return jnp.concatenate(
            [W[0:64], W[96:160], W[160:192], W[194:226], W[80:96],
             W[64:72], W[72:80], W[192:193], W[193:194],
             jnp.zeros((30, W.shape[1]), f32)], axis=0)

    W1f = permute_w1(b1_W)
    pW1 = permute_w1(b1_pW)

    x16, S, m = pl.pallas_call(
        _k1,
        grid=(_GRID,),
        in_specs=[
            _tile_spec(64), _tile_spec(64), _tile_spec(1), _tile_spec(1),
            _tile_spec(1), _tile_spec(32),
            _const_spec((64, 1024)), _const_spec((32, 1024)),
            _const_spec((1024, 32)), _const_spec((32, 32)),
            _const_spec((1, 32)), _const_spec((1, 32)),
            _const_spec((8, 8)), _const_spec((8, 8)), _const_spec((32, 16)),
        ],
        out_specs=[
            _tile_spec(_F),
            _const_spec((_F, _F)), _const_spec((1, _F)),
        ],
        out_shape=[
            jax.ShapeDtypeStruct((_B, _F), bf16),
            jax.ShapeDtypeStruct((_F, _F), f32),
            jax.ShapeDtypeStruct((1, _F), f32),
        ],
        scratch_shapes=[pltpu.VMEM((_TILE, _F), f32)],
        compiler_params=_SEQ,
    )(u, mv, ge2, ag2, oc2, feats, T, Q, P, GW, gb, ugb, EG, emb_age, emb_occ)

    h1, y2, s2, q2 = pl.pallas_call(
        _k2,
        grid=(_GRID,),
        in_specs=[
            _tile_spec(_F), _tile_spec(32),
            _const_spec((_F, _F)), _const_spec((1, _F)),
            _const_spec((_F, 1024)),
            _const_spec((1, 1024)), _const_spec((1, 1024)),
            _const_spec((_F, 1024)), _const_spec((_F, 1024)),
            _const_spec((1, 1024)),
            _const_spec((2, 1024)), _const_spec((2, 1024)),
            _const_spec((1024, 1024)),
        ],
        out_specs=[
            _tile_spec(1024), _tile_spec(1024),
            _const_spec((1, 1024)), _const_spec((1, 1024)),
        ],
        out_shape=[
            jax.ShapeDtypeStruct((_B, 1024), bf16),
            jax.ShapeDtypeStruct((_B, 1024), f32),
            jax.ShapeDtypeStruct((1, 1024), f32),
            jax.ShapeDtypeStruct((1, 1024), f32),
        ],
        scratch_shapes=[pltpu.VMEM((1, 1024), f32), pltpu.VMEM((1, 1024), f32)],
        compiler_params=_SEQ,
    )(x16, feats, S, m, W1f, b1_g.reshape(1, 1024), b1_beta.reshape(1, 1024),
      W1f.astype(bf16), pW1.astype(bf16), b1_pb.reshape(1, 1024),
      b1_W[192:194], b1_pW[192:194], b2_W.astype(bf16))

    y3, y3p, s3, q3 = pl.pallas_call(
        _k3,
        grid=(_GRID,),
        in_specs=[
            _tile_spec(1024), _tile_spec(1024),
            _const_spec((1, 1024)), _const_spec((1, 1024)),
            _const_spec((1, 1024)), _const_spec((1, 1024)),
            _const_spec((1024, 512)), _const_spec((1024, 512)),
            _const_spec((1, 512)),
        ],
        out_specs=[
            _tile_spec(512), _tile_spec(512),
            _const_spec((1, 512)), _const_spec((1, 512)),
        ],
        out_shape=[
            jax.ShapeDtypeStruct((_B, 512), f32),
            jax.ShapeDtypeStruct((_B, 512), f32),
            jax.ShapeDtypeStruct((1, 512), f32),
            jax.ShapeDtypeStruct((1, 512), f32),
        ],
        compiler_params=_SEQ,
    )(h1, y2, s2, q2, b2_g.reshape(1, 1024), b2_beta.reshape(1, 1024),
      b3_W.astype(bf16), b3_pW.astype(bf16), b3_pb.reshape(1, 512))

    out = pl.pallas_call(
        _k4,
        grid=(_GRID,),
        in_specs=[
            _tile_spec(512), _tile_spec(512),
            _const_spec((1, 512)), _const_spec((1, 512)),
            _const_spec((1, 512)), _const_spec((1, 512)),
            _const_spec((1, 512)), _const_spec((1, 1)),
        ],
        out_specs=_tile_spec(1),
        out_shape=jax.ShapeDtypeStruct((_B, 1), f32),
        compiler_params=_SEQ,
    )(y3, y3p, s3, q3, b3_g.reshape(1, 512), b3_beta.reshape(1, 512),
      out_W.reshape(1, 512), out_b.reshape(1, 1))

    return out[:, 0]


# speedup vs baseline: 1.9189x; 1.2224x over previous
import functools

import jax
import jax.numpy as jnp
from jax import lax
from jax.experimental import pallas as pl
from jax.experimental.pallas import tpu as pltpu
from jax.experimental.pallas import tpu_sc as plsc

_B = 16384
_TILE = 512
_GRID = _B // _TILE
_F = 256
_NC = 2
_NS = 16
_BPW = _B // (_NC * _NS)


def _sc_gather3(uid, mid, ci, big, combo):
    mesh = plsc.VectorSubcoreMesh(core_axis_name="c", subcore_axis_name="s")

    def body(uid_hbm, mid_hbm, ci_hbm, big_hbm, co_hbm,
             uo_hbm, mo_hbm, co_out_hbm, idx_v, rows_v, sem):
        wid = lax.axis_index("s") * _NC + lax.axis_index("c")
        base = wid * _BPW
        pltpu.sync_copy(uid_hbm.at[pl.ds(base, _BPW)], idx_v)
        pltpu.async_copy(big_hbm.at[idx_v], rows_v, sem).wait()
        pltpu.sync_copy(rows_v, uo_hbm.at[pl.ds(base, _BPW)])
        pltpu.sync_copy(mid_hbm.at[pl.ds(base, _BPW)], idx_v)
        pltpu.async_copy(big_hbm.at[idx_v], rows_v, sem).wait()
        pltpu.sync_copy(rows_v, mo_hbm.at[pl.ds(base, _BPW)])
        pltpu.sync_copy(ci_hbm.at[pl.ds(base, _BPW)], idx_v)
        pltpu.async_copy(co_hbm.at[idx_v], rows_v, sem).wait()
        pltpu.sync_copy(rows_v, co_out_hbm.at[pl.ds(base, _BPW)])

    f = pl.kernel(
        body,
        out_type=(
            jax.ShapeDtypeStruct((_B, 128), jnp.float32),
            jax.ShapeDtypeStruct((_B, 128), jnp.float32),
            jax.ShapeDtypeStruct((_B, 128), jnp.float32),
        ),
        mesh=mesh,
        scratch_types=[
            pltpu.VMEM((_BPW,), jnp.int32),
            pltpu.VMEM((_BPW, 128), jnp.float32),
            pltpu.SemaphoreType.DMA,
        ],
    )
    return f(uid, mid, ci, big, combo)


def _k1(u_ref, mv_ref, c_ref, feats_ref,
        T_ref, Qw_ref, Pw_ref, GWw_ref, gbw_ref, ugbw_ref, W1_ref,
        x_ref, s1_ref, q1_ref, x_scr):
    i = pl.program_id(0)
    f32 = jnp.float32
    bf16 = jnp.bfloat16
    feats = feats_ref[...]
    gvw = jnp.dot(feats, GWw_ref[...], preferred_element_type=f32) + gbw_ref[...]
    u = u_ref[:, 0:64]
    tmp = jnp.dot(u.astype(bf16), T_ref[...], preferred_element_type=f32)
    gvr = jnp.dot(gvw.astype(bf16), Qw_ref[...], preferred_element_type=f32)
    cross_w = jnp.dot((tmp * gvr).astype(bf16), Pw_ref[...],
                      preferred_element_type=f32) + ugbw_ref[...]
    rt = feats[:, 19:20]
    rt_hi = rt.astype(bf16).astype(f32)
    rt_lo = rt - rt_hi
    imp = feats[:, 20:21]
    z96 = jnp.zeros((_TILE, 96), f32)
    z29 = jnp.zeros((_TILE, 29), f32)
    rtblock = jnp.concatenate([z96, rt_hi, rt_lo, imp, z29], axis=1)
    x_scr[:, 0:64] = u
    x_scr[:, 64:128] = mv_ref[:, 64:128]
    x_scr[:, 128:256] = c_ref[...] + gvw + cross_w + rtblock
    x16 = x_scr[...].astype(bf16)
    x_ref[...] = x16
    y1 = jnp.dot(x16, W1_ref[...], preferred_element_type=f32)

    @pl.when(i == 0)
    def _():
        s1_ref[...] = jnp.zeros_like(s1_ref)
        q1_ref[...] = jnp.zeros_like(q1_ref)

    s1_ref[...] += jnp.sum(y1, axis=0, keepdims=True)
    q1_ref[...] += jnp.sum(y1 * y1, axis=0, keepdims=True)


def _bn_scale_shift(s_ref, q_ref, g_ref, be_ref):
    mu = s_ref[...] * (1.0 / _B)
    var = q_ref[...] * (1.0 / _B) - mu * mu
    scale = lax.rsqrt(var + 1e-5) * g_ref[...]
    shift = be_ref[...] - mu * scale
    return scale, shift


def _k2(x_ref, s1_ref, q1_ref, g_ref, be_ref,
        W1_ref, pW1_ref, pb1_ref, W2_ref,
        h1_ref, y2_ref, s2_ref, q2_ref):
    i = pl.program_id(0)
    f32 = jnp.float32
    bf16 = jnp.bfloat16
    scale, shift = _bn_scale_shift(s1_ref, q1_ref, g_ref, be_ref)
    x = x_ref[...]
    y1 = jnp.dot(x, W1_ref[...], preferred_element_type=f32)
    y1p = jnp.dot(x, pW1_ref[...], preferred_element_type=f32) + pb1_ref[...]
    h1 = jnp.maximum(y1 * scale + shift, 0.0) + y1p
    h1b = h1.astype(bf16)
    h1_ref[...] = h1b
    y2 = jnp.dot(h1b, W2_ref[...], preferred_element_type=f32)
    y2b = y2.astype(bf16)
    y2_ref[...] = y2b
    y2f = y2b.astype(f32)

    @pl.when(i == 0)
    def _():
        s2_ref[...] = jnp.zeros_like(s2_ref)
        q2_ref[...] = jnp.zeros_like(q2_ref)

    s2_ref[...] += jnp.sum(y2f, axis=0, keepdims=True)
    q2_ref[...] += jnp.sum(y2f * y2f, axis=0, keepdims=True)


def _k3(h1_ref, y2_ref, s_ref, q_ref, g_ref, be_ref, W3_ref, pW3_ref, pb3_ref,
        ow_ref, y3_ref, lo3_ref, s3_ref, q3_ref):
    i = pl.program_id(0)
    f32 = jnp.float32
    bf16 = jnp.bfloat16
    scale, shift = _bn_scale_shift(s_ref, q_ref, g_ref, be_ref)
    h2 = (jnp.maximum(y2_ref[...].astype(f32) * scale + shift, 0.0)
          + h1_ref[...].astype(f32))
    h2b = h2.astype(bf16)
    y3 = jnp.dot(h2b, W3_ref[...], preferred_element_type=f32)
    y3p = jnp.dot(h2b, pW3_ref[...], preferred_element_type=f32) + pb3_ref[...]
    y3b = y3.astype(bf16)
    y3_ref[...] = y3b
    y3f = y3b.astype(f32)
    lo3_ref[...] = jnp.sum(y3p * ow_ref[...], axis=1, keepdims=True)

    @pl.when(i == 0)
    def _():
        s3_ref[...] = jnp.zeros_like(s3_ref)
        q3_ref[...] = jnp.zeros_like(q3_ref)

    s3_ref[...] += jnp.sum(y3f, axis=0, keepdims=True)
    q3_ref[...] += jnp.sum(y3f * y3f, axis=0, keepdims=True)


def _k4(y3_ref, lo3_ref, s_ref, q_ref, g_ref, be_ref, ow_ref, ob_ref, out_ref):
    scale, shift = _bn_scale_shift(s_ref, q_ref, g_ref, be_ref)
    h3r = jnp.maximum(y3_ref[...].astype(jnp.float32) * scale + shift, 0.0)
    out_ref[...] = (jnp.sum(h3r * ow_ref[...], axis=1, keepdims=True)
                    + lo3_ref[...] + ob_ref[...])


def _const_spec(shape):
    nd = len(shape)
    return pl.BlockSpec(shape, lambda i: (0,) * nd)


def _tile_spec(cols):
    return pl.BlockSpec((_TILE, cols), lambda i: (i, 0))


_SEQ = pltpu.CompilerParams(dimension_semantics=("arbitrary",))


def kernel(user_id, gender, age, occupation, movie_id, genres, rating, implicit,
           emb_user, emb_gender, emb_age, emb_occ, emb_movie,
           genre_W, genre_b, ug_W, ug_b,
           b1_W, b1_b, b1_g, b1_beta, b1_pW, b1_pb,
           b2_W, b2_b, b2_g, b2_beta,
           b3_W, b3_b, b3_g, b3_beta, b3_pW, b3_pb,
           out_W, out_b):
    f32 = jnp.float32
    bf16 = jnp.bfloat16
    uid = user_id.astype(jnp.int32)
    mid = movie_id.astype(jnp.int32)
    ci = (gender * 256 + age * 32 + occupation).astype(jnp.int32)

    big = jnp.concatenate([emb_user, emb_movie], axis=1)
    g1024 = jnp.arange(1024)
    combo = jnp.concatenate(
        [emb_gender[g1024 // 256], emb_age[(g1024 // 32) % 8],
         emb_occ[g1024 % 32], jnp.zeros((1024, 96), f32)], axis=1)

    u128, m128, c128 = _sc_gather3(uid, mid, ci, big, combo)

    feats = jnp.concatenate(
        [genres, rating[:, None], implicit[:, None],
         jnp.zeros((_B, 11), f32)], axis=1)

    T = ug_W.reshape(64, 1024).astype(bf16)
    c1024 = jnp.arange(1024)
    Qw = jnp.zeros((128, 1024), bf16).at[32:64].set(
        (jnp.arange(32)[:, None] == (c1024[None, :] // 32)).astype(bf16))
    Pw = jnp.zeros((1024, 128), bf16).at[:, 64:96].set(
        ((c1024[:, None] % 32) == jnp.arange(32)[None, :]).astype(bf16))
    GWw = jnp.zeros((32, 128), f32).at[0:19, 32:64].set(genre_W)
    gbw = jnp.zeros((1, 128), f32).at[0, 32:64].set(genre_b)
    ugbw = jnp.zeros((1, 128), f32).at[0, 64:96].set(ug_b)

    def permute_w1(W):
        return jnp.concatenate(
            [W[0:64], W[96:160], W[64:96], W[160:192], W[194:226],
             W[192:193], W[192:193], W[193:194],
             jnp.zeros((29, W.shape[1]), f32)], axis=0)

    W1b = permute_w1(b1_W).astype(bf16)
    pW1b = permute_w1(b1_pW).astype(bf16)

    x16, s1, q1 = pl.pallas_call(
        _k1,
        grid=(_GRID,),
        in_specs=[
            _tile_spec(128), _tile_spec(128), _tile_spec(128), _tile_spec(32),
            _const_spec((64, 1024)), _const_spec((128, 1024)),
            _const_spec((1024, 128)), _const_spec((32, 128)),
            _const_spec((1, 128)), _const_spec((1, 128)),
            _const_spec((_F, 1024)),
        ],
        out_specs=[
            _tile_spec(_F),
            _const_spec((1, 1024)), _const_spec((1, 1024)),
        ],
        out_shape=[
            jax.ShapeDtypeStruct((_B, _F), bf16),
            jax.ShapeDtypeStruct((1, 1024), f32),
            jax.ShapeDtypeStruct((1, 1024), f32),
        ],
        scratch_shapes=[pltpu.VMEM((_TILE, _F), f32)],
        compiler_params=_SEQ,
    )(u128, m128, c128, feats, T, Qw, Pw, GWw, gbw, ugbw, W1b)

    h1, y2, s2, q2 = pl.pallas_call(
        _k2,
        grid=(_GRID,),
        in_specs=[
            _tile_spec(_F),
            _const_spec((1, 1024)), _const_spec((1, 1024)),
            _const_spec((1, 1024)), _const_spec((1, 1024)),
            _const_spec((_F, 1024)), _const_spec((_F, 1024)),
            _const_spec((1, 1024)), _const_spec((1024, 1024)),
        ],
        out_specs=[
            _tile_spec(1024), _tile_spec(1024),
            _const_spec((1, 1024)), _const_spec((1, 1024)),
        ],
        out_shape=[
            jax.ShapeDtypeStruct((_B, 1024), bf16),
            jax.ShapeDtypeStruct((_B, 1024), bf16),
            jax.ShapeDtypeStruct((1, 1024), f32),
            jax.ShapeDtypeStruct((1, 1024), f32),
        ],
        compiler_params=_SEQ,
    )(x16, s1, q1, b1_g.reshape(1, 1024), b1_beta.reshape(1, 1024),
      W1b, pW1b, b1_pb.reshape(1, 1024), b2_W.astype(bf16))

    ow = out_W.reshape(1, 512)
    y3, lo3, s3, q3 = pl.pallas_call(
        _k3,
        grid=(_GRID,),
        in_specs=[
            _tile_spec(1024), _tile_spec(1024),
            _const_spec((1, 1024)), _const_spec((1, 1024)),
            _const_spec((1, 1024)), _const_spec((1, 1024)),
            _const_spec((1024, 512)), _const_spec((1024, 512)),
            _const_spec((1, 512)), _const_spec((1, 512)),
        ],
        out_specs=[
            _tile_spec(512), _tile_spec(1),
            _const_spec((1, 512)), _const_spec((1, 512)),
        ],
        out_shape=[
            jax.ShapeDtypeStruct((_B, 512), bf16),
            jax.ShapeDtypeStruct((_B, 1), f32),
            jax.ShapeDtypeStruct((1, 512), f32),
            jax.ShapeDtypeStruct((1, 512), f32),
        ],
        compiler_params=_SEQ,
    )(h1, y2, s2, q2, b2_g.reshape(1, 1024), b2_beta.reshape(1, 1024),
      b3_W.astype(bf16), b3_pW.astype(bf16), b3_pb.reshape(1, 512), ow)

    out = pl.pallas_call(
        _k4,
        grid=(_GRID,),
        in_specs=[
            _tile_spec(512), _tile_spec(1),
            _const_spec((1, 512)), _const_spec((1, 512)),
            _const_spec((1, 512)), _const_spec((1, 512)),
            _const_spec((1, 512)), _const_spec((1, 1)),
        ],
        out_specs=_tile_spec(1),
        out_shape=jax.ShapeDtypeStruct((_B, 1), f32),
        compiler_params=_SEQ,
    )(y3, lo3, s3, q3, b3_g.reshape(1, 512), b3_beta.reshape(1, 512),
      ow, out_b.reshape(1, 1))

    return out[:, 0]


# speedup vs baseline: 2.0746x; 1.0812x over previous
import functools

import jax
import jax.numpy as jnp
from jax import lax
from jax.experimental import pallas as pl
from jax.experimental.pallas import tpu as pltpu
from jax.experimental.pallas import tpu_sc as plsc

_B = 16384
_TILE = 512
_GRID = _B // _TILE
_F = 256
_NC = 2
_NS = 16
_BPW = _B // (_NC * _NS)


def _sc_gather3(idx3, big, combo):
    mesh = plsc.VectorSubcoreMesh(core_axis_name="c", subcore_axis_name="s")

    def body(idx_hbm, big_hbm, co_hbm,
             uo_hbm, mo_hbm, co_out_hbm, idx_v, rows_v, sem):
        wid = lax.axis_index("s") * _NC + lax.axis_index("c")
        base = wid * _BPW
        pltpu.sync_copy(idx_hbm.at[pl.ds(base, _BPW)], idx_v)
        pltpu.async_copy(big_hbm.at[idx_v], rows_v, sem).wait()
        pltpu.sync_copy(rows_v, uo_hbm.at[pl.ds(base, _BPW)])
        pltpu.sync_copy(idx_hbm.at[pl.ds(_B + base, _BPW)], idx_v)
        pltpu.async_copy(big_hbm.at[idx_v], rows_v, sem).wait()
        pltpu.sync_copy(rows_v, mo_hbm.at[pl.ds(base, _BPW)])
        pltpu.sync_copy(idx_hbm.at[pl.ds(2 * _B + base, _BPW)], idx_v)
        pltpu.async_copy(co_hbm.at[idx_v], rows_v, sem).wait()
        pltpu.sync_copy(rows_v, co_out_hbm.at[pl.ds(base, _BPW)])

    f = pl.kernel(
        body,
        out_type=(
            jax.ShapeDtypeStruct((_B, 128), jnp.float32),
            jax.ShapeDtypeStruct((_B, 128), jnp.float32),
            jax.ShapeDtypeStruct((_B, 128), jnp.float32),
        ),
        mesh=mesh,
        scratch_types=[
            pltpu.VMEM((_BPW,), jnp.int32),
            pltpu.VMEM((_BPW, 128), jnp.float32),
            pltpu.SemaphoreType.DMA,
        ],
    )
    return f(idx3, big, combo)


def _k1(u_ref, mv_ref, c_ref, gen_ref, rt_ref, im_ref,
        T_ref, Qw_ref, Pw_ref, GWw_ref, gbw_ref, ugbw_ref, W1_ref,
        x_ref, s1_ref, q1_ref, x_scr):
    i = pl.program_id(0)
    f32 = jnp.float32
    bf16 = jnp.bfloat16
    gvw = jnp.dot(gen_ref[...], GWw_ref[...],
                  preferred_element_type=f32) + gbw_ref[...]
    u = u_ref[:, 0:64]
    tmp = jnp.dot(u.astype(bf16), T_ref[...], preferred_element_type=f32)
    gvr = jnp.dot(gvw.astype(bf16), Qw_ref[...], preferred_element_type=f32)
    cross_w = jnp.dot((tmp * gvr).astype(bf16), Pw_ref[...],
                      preferred_element_type=f32) + ugbw_ref[...]
    rt = rt_ref[...]
    rt_hi = rt.astype(bf16).astype(f32)
    rt_lo = rt - rt_hi
    imp = im_ref[...]
    z96 = jnp.zeros((_TILE, 96), f32)
    z29 = jnp.zeros((_TILE, 29), f32)
    rtblock = jnp.concatenate([z96, rt_hi, rt_lo, imp, z29], axis=1)
    x_scr[:, 0:64] = u
    x_scr[:, 64:128] = mv_ref[:, 64:128]
    x_scr[:, 128:256] = c_ref[...] + gvw + cross_w + rtblock
    x16 = x_scr[...].astype(bf16)
    x_ref[...] = x16
    y1 = jnp.dot(x16, W1_ref[...], preferred_element_type=f32)

    @pl.when(i == 0)
    def _():
        s1_ref[...] = jnp.zeros_like(s1_ref)
        q1_ref[...] = jnp.zeros_like(q1_ref)

    s1_ref[...] += jnp.sum(y1, axis=0, keepdims=True)
    q1_ref[...] += jnp.sum(y1 * y1, axis=0, keepdims=True)


def _bn_scale_shift(s_ref, q_ref, g_ref, be_ref):
    mu = s_ref[...] * (1.0 / _B)
    var = q_ref[...] * (1.0 / _B) - mu * mu
    scale = lax.rsqrt(var + 1e-5) * g_ref[...]
    shift = be_ref[...] - mu * scale
    return scale, shift


def _k2(x_ref, s1_ref, q1_ref, g_ref, be_ref,
        W1_ref, pW1_ref, pb1_ref, W2f_ref,
        h1_ref, y2_ref, s2_ref, q2_ref, w2_scr):
    i = pl.program_id(0)
    f32 = jnp.float32
    bf16 = jnp.bfloat16

    @pl.when(i == 0)
    def _():
        w2_scr[...] = W2f_ref[...].astype(bf16)
        s2_ref[...] = jnp.zeros_like(s2_ref)
        q2_ref[...] = jnp.zeros_like(q2_ref)

    scale, shift = _bn_scale_shift(s1_ref, q1_ref, g_ref, be_ref)
    x = x_ref[...]
    y1 = jnp.dot(x, W1_ref[...], preferred_element_type=f32)
    y1p = jnp.dot(x, pW1_ref[...], preferred_element_type=f32) + pb1_ref[...]
    h1 = jnp.maximum(y1 * scale + shift, 0.0) + y1p
    h1b = h1.astype(bf16)
    h1_ref[...] = h1b
    y2 = jnp.dot(h1b, w2_scr[...], preferred_element_type=f32)
    y2b = y2.astype(bf16)
    y2_ref[...] = y2b
    y2f = y2b.astype(f32)

    s2_ref[...] += jnp.sum(y2f, axis=0, keepdims=True)
    q2_ref[...] += jnp.sum(y2f * y2f, axis=0, keepdims=True)


def _k3(h1_ref, y2_ref, s_ref, q_ref, g_ref, be_ref, W3f_ref, pW3f_ref,
        pb3_ref, ow_ref, y3_ref, lo3_ref, s3_ref, q3_ref, w3_scr, pw3_scr):
    i = pl.program_id(0)
    f32 = jnp.float32
    bf16 = jnp.bfloat16

    @pl.when(i == 0)
    def _():
        w3_scr[...] = W3f_ref[...].astype(bf16)
        pw3_scr[...] = pW3f_ref[...].astype(bf16)
        s3_ref[...] = jnp.zeros_like(s3_ref)
        q3_ref[...] = jnp.zeros_like(q3_ref)

    scale, shift = _bn_scale_shift(s_ref, q_ref, g_ref, be_ref)
    h2 = (jnp.maximum(y2_ref[...].astype(f32) * scale + shift, 0.0)
          + h1_ref[...].astype(f32))
    h2b = h2.astype(bf16)
    y3 = jnp.dot(h2b, w3_scr[...], preferred_element_type=f32)
    y3p = jnp.dot(h2b, pw3_scr[...], preferred_element_type=f32) + pb3_ref[...]
    y3b = y3.astype(bf16)
    y3_ref[...] = y3b
    y3f = y3b.astype(f32)
    lo3_ref[...] = jnp.sum(y3p * ow_ref[...], axis=1, keepdims=True)

    s3_ref[...] += jnp.sum(y3f, axis=0, keepdims=True)
    q3_ref[...] += jnp.sum(y3f * y3f, axis=0, keepdims=True)


def _k4(y3_ref, lo3_ref, s_ref, q_ref, g_ref, be_ref, ow_ref, ob_ref, out_ref):
    scale, shift = _bn_scale_shift(s_ref, q_ref, g_ref, be_ref)
    h3r = jnp.maximum(y3_ref[...].astype(jnp.float32) * scale + shift, 0.0)
    out_ref[...] = (jnp.sum(h3r * ow_ref[...], axis=1, keepdims=True)
                    + lo3_ref[...] + ob_ref[...])


def _const_spec(shape):
    nd = len(shape)
    return pl.BlockSpec(shape, lambda i: (0,) * nd)


def _tile_spec(cols, rows=_TILE):
    return pl.BlockSpec((rows, cols), lambda i: (i, 0))


_SEQ = pltpu.CompilerParams(dimension_semantics=("arbitrary",))


def kernel(user_id, gender, age, occupation, movie_id, genres, rating, implicit,
           emb_user, emb_gender, emb_age, emb_occ, emb_movie,
           genre_W, genre_b, ug_W, ug_b,
           b1_W, b1_b, b1_g, b1_beta, b1_pW, b1_pb,
           b2_W, b2_b, b2_g, b2_beta,
           b3_W, b3_b, b3_g, b3_beta, b3_pW, b3_pb,
           out_W, out_b):
    f32 = jnp.float32
    bf16 = jnp.bfloat16
    idx3 = jnp.concatenate([
        user_id.astype(jnp.int32),
        movie_id.astype(jnp.int32),
        (gender * 256 + age * 32 + occupation).astype(jnp.int32),
    ])

    big = jnp.concatenate([emb_user, emb_movie], axis=1)
    g1024 = jnp.arange(1024)
    combo = jnp.concatenate(
        [emb_gender[g1024 // 256], emb_age[(g1024 // 32) % 8],
         emb_occ[g1024 % 32], jnp.zeros((1024, 96), f32)], axis=1)

    u128, m128, c128 = _sc_gather3(idx3, big, combo)

    T = ug_W.reshape(64, 1024).astype(bf16)
    c1024 = jnp.arange(1024)
    Qw = jnp.pad((jnp.arange(32)[:, None] == (c1024[None, :] // 32)
                  ).astype(bf16), ((32, 64), (0, 0)))
    Pw = jnp.pad(((c1024[:, None] % 32) == jnp.arange(32)[None, :]
                  ).astype(bf16), ((0, 0), (64, 32)))
    GWw = jnp.pad(genre_W, ((0, 0), (32, 64)))
    gbw = jnp.pad(genre_b[None, :], ((0, 0), (32, 64)))
    ugbw = jnp.pad(ug_b[None, :], ((0, 0), (64, 32)))

    def permute_w1(W):
        return jnp.concatenate(
            [W[0:64], W[96:160], W[64:96], W[160:192], W[194:226],
             W[192:193], W[192:193], W[193:194],
             jnp.zeros((29, W.shape[1]), f32)], axis=0)

    W1b = permute_w1(b1_W).astype(bf16)
    pW1b = permute_w1(b1_pW).astype(bf16)

    x16, s1, q1 = pl.pallas_call(
        _k1,
        grid=(_GRID,),
        in_specs=[
            _tile_spec(128), _tile_spec(128), _tile_spec(128),
            _tile_spec(19), _tile_spec(1), _tile_spec(1),
            _const_spec((64, 1024)), _const_spec((128, 1024)),
            _const_spec((1024, 128)), _const_spec((19, 128)),
            _const_spec((1, 128)), _const_spec((1, 128)),
            _const_spec((_F, 1024)),
        ],
        out_specs=[
            _tile_spec(_F),
            _const_spec((1, 1024)), _const_spec((1, 1024)),
        ],
        out_shape=[
            jax.ShapeDtypeStruct((_B, _F), bf16),
            jax.ShapeDtypeStruct((1, 1024), f32),
            jax.ShapeDtypeStruct((1, 1024), f32),
        ],
        scratch_shapes=[pltpu.VMEM((_TILE, _F), f32)],
        compiler_params=_SEQ,
    )(u128, m128, c128, genres, rating[:, None], implicit[:, None],
      T, Qw, Pw, GWw, gbw, ugbw, W1b)

    h1, y2, s2, q2 = pl.pallas_call(
        _k2,
        grid=(_GRID,),
        in_specs=[
            _tile_spec(_F),
            _const_spec((1, 1024)), _const_spec((1, 1024)),
            _const_spec((1, 1024)), _const_spec((1, 1024)),
            _const_spec((_F, 1024)), _const_spec((_F, 1024)),
            _const_spec((1, 1024)), _const_spec((1024, 1024)),
        ],
        out_specs=[
            _tile_spec(1024), _tile_spec(1024),
            _const_spec((1, 1024)), _const_spec((1, 1024)),
        ],
        out_shape=[
            jax.ShapeDtypeStruct((_B, 1024), bf16),
            jax.ShapeDtypeStruct((_B, 1024), bf16),
            jax.ShapeDtypeStruct((1, 1024), f32),
            jax.ShapeDtypeStruct((1, 1024), f32),
        ],
        scratch_shapes=[pltpu.VMEM((1024, 1024), bf16)],
        compiler_params=_SEQ,
    )(x16, s1, q1, b1_g.reshape(1, 1024), b1_beta.reshape(1, 1024),
      W1b, pW1b, b1_pb.reshape(1, 1024), b2_W)

    ow = out_W.reshape(1, 512)
    t3 = 1024
    y3, lo3, s3, q3 = pl.pallas_call(
        _k3,
        grid=(_B // t3,),
        in_specs=[
            _tile_spec(1024, t3), _tile_spec(1024, t3),
            _const_spec((1, 1024)), _const_spec((1, 1024)),
            _const_spec((1, 1024)), _const_spec((1, 1024)),
            _const_spec((1024, 512)), _const_spec((1024, 512)),
            _const_spec((1, 512)), _const_spec((1, 512)),
        ],
        out_specs=[
            _tile_spec(512, t3), _tile_spec(1, t3),
            _const_spec((1, 512)), _const_spec((1, 512)),
        ],
        out_shape=[
            jax.ShapeDtypeStruct((_B, 512), bf16),
            jax.ShapeDtypeStruct((_B, 1), f32),
            jax.ShapeDtypeStruct((1, 512), f32),
            jax.ShapeDtypeStruct((1, 512), f32),
        ],
        scratch_shapes=[pltpu.VMEM((1024, 512), bf16),
                        pltpu.VMEM((1024, 512), bf16)],
        compiler_params=_SEQ,
    )(h1, y2, s2, q2, b2_g.reshape(1, 1024), b2_beta.reshape(1, 1024),
      b3_W, b3_pW, b3_pb.reshape(1, 512), ow)

    t4 = 2048
    out = pl.pallas_call(
        _k4,
        grid=(_B // t4,),
        in_specs=[
            _tile_spec(512, t4), _tile_spec(1, t4),
            _const_spec((1, 512)), _const_spec((1, 512)),
            _const_spec((1, 512)), _const_spec((1, 512)),
            _const_spec((1, 512)), _const_spec((1, 1)),
        ],
        out_specs=_tile_spec(1, t4),
        out_shape=jax.ShapeDtypeStruct((_B, 1), f32),
        compiler_params=_SEQ,
    )(y3, lo3, s3, q3, b3_g.reshape(1, 512), b3_beta.reshape(1, 512),
      ow, out_b.reshape(1, 1))

    return out[:, 0]


# speedup vs baseline: 2.1034x; 1.0139x over previous
import functools

import jax
import jax.numpy as jnp
from jax import lax
from jax.experimental import pallas as pl
from jax.experimental.pallas import tpu as pltpu
from jax.experimental.pallas import tpu_sc as plsc

_B = 16384
_TILE = 1024
_GRID = _B // _TILE
_F = 256
_NC = 2
_NS = 16
_BPW = _B // (_NC * _NS)


def _sc_gather3(idx3, big, combo):
    mesh = plsc.VectorSubcoreMesh(core_axis_name="c", subcore_axis_name="s")

    def body(idx_hbm, big_hbm, co_hbm,
             uo_hbm, mo_hbm, co_out_hbm, idx_v, rows_v, sem):
        wid = lax.axis_index("s") * _NC + lax.axis_index("c")
        base = wid * _BPW
        pltpu.sync_copy(idx_hbm.at[pl.ds(base, _BPW)], idx_v)
        pltpu.async_copy(big_hbm.at[idx_v], rows_v, sem).wait()
        pltpu.sync_copy(rows_v, uo_hbm.at[pl.ds(base, _BPW)])
        pltpu.sync_copy(idx_hbm.at[pl.ds(_B + base, _BPW)], idx_v)
        pltpu.async_copy(big_hbm.at[idx_v], rows_v, sem).wait()
        pltpu.sync_copy(rows_v, mo_hbm.at[pl.ds(base, _BPW)])
        pltpu.sync_copy(idx_hbm.at[pl.ds(2 * _B + base, _BPW)], idx_v)
        pltpu.async_copy(co_hbm.at[idx_v], rows_v, sem).wait()
        pltpu.sync_copy(rows_v, co_out_hbm.at[pl.ds(base, _BPW)])

    f = pl.kernel(
        body,
        out_type=(
            jax.ShapeDtypeStruct((_B, 128), jnp.float32),
            jax.ShapeDtypeStruct((_B, 128), jnp.float32),
            jax.ShapeDtypeStruct((_B, 128), jnp.float32),
        ),
        mesh=mesh,
        scratch_types=[
            pltpu.VMEM((_BPW,), jnp.int32),
            pltpu.VMEM((_BPW, 128), jnp.float32),
            pltpu.SemaphoreType.DMA,
        ],
    )
    return f(idx3, big, combo)


def _k1(u_ref, mv_ref, c_ref, gen_ref, rt_ref, im_ref,
        T_ref, Qw_ref, Pw_ref, GWw_ref, gbw_ref, ugbw_ref, W1_ref,
        x_ref, s1_ref, q1_ref, x_scr):
    i = pl.program_id(0)
    f32 = jnp.float32
    bf16 = jnp.bfloat16
    gvw = jnp.dot(gen_ref[...], GWw_ref[...],
                  preferred_element_type=f32) + gbw_ref[...]
    u = u_ref[:, 0:64]
    tmp = jnp.dot(u.astype(bf16), T_ref[...], preferred_element_type=f32)
    gvr = jnp.dot(gvw.astype(bf16), Qw_ref[...], preferred_element_type=f32)
    cross_w = jnp.dot((tmp * gvr).astype(bf16), Pw_ref[...],
                      preferred_element_type=f32) + ugbw_ref[...]
    rt = rt_ref[...]
    rt_hi = rt.astype(bf16).astype(f32)
    rt_lo = rt - rt_hi
    imp = im_ref[...]
    z96 = jnp.zeros((_TILE, 96), f32)
    z29 = jnp.zeros((_TILE, 29), f32)
    rtblock = jnp.concatenate([z96, rt_hi, rt_lo, imp, z29], axis=1)
    x_scr[:, 0:64] = u
    x_scr[:, 64:128] = mv_ref[:, 64:128]
    x_scr[:, 128:256] = c_ref[...] + gvw + cross_w + rtblock
    x16 = x_scr[...].astype(bf16)
    x_ref[...] = x16
    y1 = jnp.dot(x16, W1_ref[...], preferred_element_type=f32)

    @pl.when(i == 0)
    def _():
        s1_ref[...] = jnp.zeros_like(s1_ref)
        q1_ref[...] = jnp.zeros_like(q1_ref)

    s1_ref[...] += jnp.sum(y1, axis=0, keepdims=True)
    q1_ref[...] += jnp.sum(y1 * y1, axis=0, keepdims=True)


def _bn_scale_shift(s_ref, q_ref, g_ref, be_ref):
    mu = s_ref[...] * (1.0 / _B)
    var = q_ref[...] * (1.0 / _B) - mu * mu
    scale = lax.rsqrt(var + 1e-5) * g_ref[...]
    shift = be_ref[...] - mu * scale
    return scale, shift


def _k2(x_ref, s1_ref, q1_ref, g_ref, be_ref,
        W1_ref, pW1_ref, pb1_ref, W2f_ref,
        h1_ref, y2_ref, s2_ref, q2_ref, w2_scr):
    i = pl.program_id(0)
    f32 = jnp.float32
    bf16 = jnp.bfloat16

    @pl.when(i == 0)
    def _():
        w2_scr[...] = W2f_ref[...].astype(bf16)
        s2_ref[...] = jnp.zeros_like(s2_ref)
        q2_ref[...] = jnp.zeros_like(q2_ref)

    scale, shift = _bn_scale_shift(s1_ref, q1_ref, g_ref, be_ref)
    x = x_ref[...]
    y1 = jnp.dot(x, W1_ref[...], preferred_element_type=f32)
    y1p = jnp.dot(x, pW1_ref[...], preferred_element_type=f32) + pb1_ref[...]
    h1 = jnp.maximum(y1 * scale + shift, 0.0) + y1p
    h1b = h1.astype(bf16)
    h1_ref[...] = h1b
    y2 = jnp.dot(h1b, w2_scr[...], preferred_element_type=f32)
    y2b = y2.astype(bf16)
    y2_ref[...] = y2b
    y2f = y2b.astype(f32)

    s2_ref[...] += jnp.sum(y2f, axis=0, keepdims=True)
    q2_ref[...] += jnp.sum(y2f * y2f, axis=0, keepdims=True)


def _k3(h1_ref, y2_ref, s_ref, q_ref, g_ref, be_ref, W3f_ref, pW3f_ref,
        pb3_ref, ow_ref, y3_ref, lo3_ref, s3_ref, q3_ref, w3_scr, pw3_scr):
    i = pl.program_id(0)
    f32 = jnp.float32
    bf16 = jnp.bfloat16

    @pl.when(i == 0)
    def _():
        w3_scr[...] = W3f_ref[...].astype(bf16)
        pw3_scr[...] = pW3f_ref[...].astype(bf16)
        s3_ref[...] = jnp.zeros_like(s3_ref)
        q3_ref[...] = jnp.zeros_like(q3_ref)

    scale, shift = _bn_scale_shift(s_ref, q_ref, g_ref, be_ref)
    h2 = (jnp.maximum(y2_ref[...].astype(f32) * scale + shift, 0.0)
          + h1_ref[...].astype(f32))
    h2b = h2.astype(bf16)
    y3 = jnp.dot(h2b, w3_scr[...], preferred_element_type=f32)
    y3p = jnp.dot(h2b, pw3_scr[...], preferred_element_type=f32) + pb3_ref[...]
    y3b = y3.astype(bf16)
    y3_ref[...] = y3b
    y3f = y3b.astype(f32)
    lo3_ref[...] = jnp.sum(y3p * ow_ref[...], axis=1, keepdims=True)

    s3_ref[...] += jnp.sum(y3f, axis=0, keepdims=True)
    q3_ref[...] += jnp.sum(y3f * y3f, axis=0, keepdims=True)


def _k4(y3_ref, lo3_ref, s_ref, q_ref, g_ref, be_ref, ow_ref, ob_ref, out_ref):
    scale, shift = _bn_scale_shift(s_ref, q_ref, g_ref, be_ref)
    h3r = jnp.maximum(y3_ref[...].astype(jnp.float32) * scale + shift, 0.0)
    out_ref[...] = (jnp.sum(h3r * ow_ref[...], axis=1, keepdims=True)
                    + lo3_ref[...] + ob_ref[...])


def _const_spec(shape):
    nd = len(shape)
    return pl.BlockSpec(shape, lambda i: (0,) * nd)


def _tile_spec(cols, rows=_TILE):
    return pl.BlockSpec((rows, cols), lambda i: (i, 0))


_SEQ = pltpu.CompilerParams(dimension_semantics=("arbitrary",))


def kernel(user_id, gender, age, occupation, movie_id, genres, rating, implicit,
           emb_user, emb_gender, emb_age, emb_occ, emb_movie,
           genre_W, genre_b, ug_W, ug_b,
           b1_W, b1_b, b1_g, b1_beta, b1_pW, b1_pb,
           b2_W, b2_b, b2_g, b2_beta,
           b3_W, b3_b, b3_g, b3_beta, b3_pW, b3_pb,
           out_W, out_b):
    f32 = jnp.float32
    bf16 = jnp.bfloat16
    idx3 = jnp.concatenate([
        user_id.astype(jnp.int32),
        movie_id.astype(jnp.int32),
        (gender * 256 + age * 32 + occupation).astype(jnp.int32),
    ])

    big = jnp.concatenate([emb_user, emb_movie], axis=1)
    g1024 = jnp.arange(1024)
    combo = jnp.concatenate(
        [emb_gender[g1024 // 256], emb_age[(g1024 // 32) % 8],
         emb_occ[g1024 % 32], jnp.zeros((1024, 96), f32)], axis=1)

    u128, m128, c128 = _sc_gather3(idx3, big, combo)

    T = ug_W.reshape(64, 1024).astype(bf16)
    c1024 = jnp.arange(1024)
    Qw = jnp.pad((jnp.arange(32)[:, None] == (c1024[None, :] // 32)
                  ).astype(bf16), ((32, 64), (0, 0)))
    Pw = jnp.pad(((c1024[:, None] % 32) == jnp.arange(32)[None, :]
                  ).astype(bf16), ((0, 0), (64, 32)))
    GWw = jnp.pad(genre_W, ((0, 0), (32, 64)))
    gbw = jnp.pad(genre_b[None, :], ((0, 0), (32, 64)))
    ugbw = jnp.pad(ug_b[None, :], ((0, 0), (64, 32)))

    def permute_w1(W):
        return jnp.concatenate(
            [W[0:64], W[96:160], W[64:96], W[160:192], W[194:226],
             W[192:193], W[192:193], W[193:194],
             jnp.zeros((29, W.shape[1]), f32)], axis=0)

    W1b = permute_w1(b1_W).astype(bf16)
    pW1b = permute_w1(b1_pW).astype(bf16)

    x16, s1, q1 = pl.pallas_call(
        _k1,
        grid=(_GRID,),
        in_specs=[
            _tile_spec(128), _tile_spec(128), _tile_spec(128),
            _tile_spec(19), _tile_spec(1), _tile_spec(1),
            _const_spec((64, 1024)), _const_spec((128, 1024)),
            _const_spec((1024, 128)), _const_spec((19, 128)),
            _const_spec((1, 128)), _const_spec((1, 128)),
            _const_spec((_F, 1024)),
        ],
        out_specs=[
            _tile_spec(_F),
            _const_spec((1, 1024)), _const_spec((1, 1024)),
        ],
        out_shape=[
            jax.ShapeDtypeStruct((_B, _F), bf16),
            jax.ShapeDtypeStruct((1, 1024), f32),
            jax.ShapeDtypeStruct((1, 1024), f32),
        ],
        scratch_shapes=[pltpu.VMEM((_TILE, _F), f32)],
        compiler_params=_SEQ,
    )(u128, m128, c128, genres, rating[:, None], implicit[:, None],
      T, Qw, Pw, GWw, gbw, ugbw, W1b)

    h1, y2, s2, q2 = pl.pallas_call(
        _k2,
        grid=(_GRID,),
        in_specs=[
            _tile_spec(_F),
            _const_spec((1, 1024)), _const_spec((1, 1024)),
            _const_spec((1, 1024)), _const_spec((1, 1024)),
            _const_spec((_F, 1024)), _const_spec((_F, 1024)),
            _const_spec((1, 1024)), _const_spec((1024, 1024)),
        ],
        out_specs=[
            _tile_spec(1024), _tile_spec(1024),
            _const_spec((1, 1024)), _const_spec((1, 1024)),
        ],
        out_shape=[
            jax.ShapeDtypeStruct((_B, 1024), bf16),
            jax.ShapeDtypeStruct((_B, 1024), bf16),
            jax.ShapeDtypeStruct((1, 1024), f32),
            jax.ShapeDtypeStruct((1, 1024), f32),
        ],
        scratch_shapes=[pltpu.VMEM((1024, 1024), bf16)],
        compiler_params=_SEQ,
    )(x16, s1, q1, b1_g.reshape(1, 1024), b1_beta.reshape(1, 1024),
      W1b, pW1b, b1_pb.reshape(1, 1024), b2_W)

    ow = out_W.reshape(1, 512)
    t3 = 1024
    y3, lo3, s3, q3 = pl.pallas_call(
        _k3,
        grid=(_B // t3,),
        in_specs=[
            _tile_spec(1024, t3), _tile_spec(1024, t3),
            _const_spec((1, 1024)), _const_spec((1, 1024)),
            _const_spec((1, 1024)), _const_spec((1, 1024)),
            _const_spec((1024, 512)), _const_spec((1024, 512)),
            _const_spec((1, 512)), _const_spec((1, 512)),
        ],
        out_specs=[
            _tile_spec(512, t3), _tile_spec(1, t3),
            _const_spec((1, 512)), _const_spec((1, 512)),
        ],
        out_shape=[
            jax.ShapeDtypeStruct((_B, 512), bf16),
            jax.ShapeDtypeStruct((_B, 1), f32),
            jax.ShapeDtypeStruct((1, 512), f32),
            jax.ShapeDtypeStruct((1, 512), f32),
        ],
        scratch_shapes=[pltpu.VMEM((1024, 512), bf16),
                        pltpu.VMEM((1024, 512), bf16)],
        compiler_params=_SEQ,
    )(h1, y2, s2, q2, b2_g.reshape(1, 1024), b2_beta.reshape(1, 1024),
      b3_W, b3_pW, b3_pb.reshape(1, 512), ow)

    t4 = 2048
    out = pl.pallas_call(
        _k4,
        grid=(_B // t4,),
        in_specs=[
            _tile_spec(512, t4), _tile_spec(1, t4),
            _const_spec((1, 512)), _const_spec((1, 512)),
            _const_spec((1, 512)), _const_spec((1, 512)),
            _const_spec((1, 512)), _const_spec((1, 1)),
        ],
        out_specs=_tile_spec(1, t4),
        out_shape=jax.ShapeDtypeStruct((_B, 1), f32),
        compiler_params=_SEQ,
    )(y3, lo3, s3, q3, b3_g.reshape(1, 512), b3_beta.reshape(1, 512),
      ow, out_b.reshape(1, 1))

    return out[:, 0]


# speedup vs baseline: 2.1157x; 1.0058x over previous
import functools

import jax
import jax.numpy as jnp
from jax import lax
from jax.experimental import pallas as pl
from jax.experimental.pallas import tpu as pltpu
from jax.experimental.pallas import tpu_sc as plsc

_B = 16384
_TILE = 1024
_GRID = _B // _TILE
_F = 256
_NC = 2
_NS = 16
_BPW = _B // (_NC * _NS)


def _sc_gather3(idx3, big, combo):
    mesh = plsc.VectorSubcoreMesh(core_axis_name="c", subcore_axis_name="s")

    def body(idx_hbm, big_hbm, co_hbm,
             uo_hbm, mo_hbm, co_out_hbm, idx_v, rows_v, sem):
        wid = lax.axis_index("s") * _NC + lax.axis_index("c")
        base = wid * _BPW
        pltpu.sync_copy(idx_hbm.at[pl.ds(base, _BPW)], idx_v)
        pltpu.async_copy(big_hbm.at[idx_v], rows_v, sem).wait()
        pltpu.sync_copy(rows_v, uo_hbm.at[pl.ds(base, _BPW)])
        pltpu.sync_copy(idx_hbm.at[pl.ds(_B + base, _BPW)], idx_v)
        pltpu.async_copy(big_hbm.at[idx_v], rows_v, sem).wait()
        pltpu.sync_copy(rows_v, mo_hbm.at[pl.ds(base, _BPW)])
        pltpu.sync_copy(idx_hbm.at[pl.ds(2 * _B + base, _BPW)], idx_v)
        pltpu.async_copy(co_hbm.at[idx_v], rows_v, sem).wait()
        pltpu.sync_copy(rows_v, co_out_hbm.at[pl.ds(base, _BPW)])

    f = pl.kernel(
        body,
        out_type=(
            jax.ShapeDtypeStruct((_B, 128), jnp.float32),
            jax.ShapeDtypeStruct((_B, 128), jnp.float32),
            jax.ShapeDtypeStruct((_B, 128), jnp.float32),
        ),
        mesh=mesh,
        scratch_types=[
            pltpu.VMEM((_BPW,), jnp.int32),
            pltpu.VMEM((_BPW, 128), jnp.float32),
            pltpu.SemaphoreType.DMA,
        ],
    )
    return f(idx3, big, combo)


def _k1(u_ref, mv_ref, c_ref, gen_ref, rt_ref, im_ref,
        T_ref, Qw_ref, Pw_ref, GWw_ref, gbw_ref, ugbw_ref, W1_ref,
        x_ref, s1_ref, q1_ref):
    i = pl.program_id(0)
    f32 = jnp.float32
    bf16 = jnp.bfloat16
    gvw = jnp.dot(gen_ref[...], GWw_ref[...],
                  preferred_element_type=f32) + gbw_ref[...]
    u = u_ref[:, 0:64]
    tmp = jnp.dot(u.astype(bf16), T_ref[...], preferred_element_type=f32)
    gvr = jnp.dot(gvw.astype(bf16), Qw_ref[...], preferred_element_type=f32)
    cross_w = jnp.dot((tmp * gvr).astype(bf16), Pw_ref[...],
                      preferred_element_type=f32) + ugbw_ref[...]
    rt = rt_ref[...]
    rt_hi = rt.astype(bf16).astype(f32)
    rt_lo = rt - rt_hi
    imp = im_ref[...]
    z96 = jnp.zeros((_TILE, 96), f32)
    z29 = jnp.zeros((_TILE, 29), f32)
    rtblock = jnp.concatenate([z96, rt_hi, rt_lo, imp, z29], axis=1)
    reg1 = c_ref[...] + gvw + cross_w + rtblock
    x16 = jnp.concatenate([u, mv_ref[:, 64:128], reg1], axis=1).astype(bf16)
    x_ref[...] = x16
    y1 = jnp.dot(x16, W1_ref[...], preferred_element_type=f32)

    @pl.when(i == 0)
    def _():
        s1_ref[...] = jnp.zeros_like(s1_ref)
        q1_ref[...] = jnp.zeros_like(q1_ref)

    s1_ref[...] += jnp.sum(y1, axis=0, keepdims=True)
    q1_ref[...] += jnp.sum(y1 * y1, axis=0, keepdims=True)


def _bn_scale_shift(s_ref, q_ref, g_ref, be_ref):
    mu = s_ref[...] * (1.0 / _B)
    var = q_ref[...] * (1.0 / _B) - mu * mu
    scale = lax.rsqrt(var + 1e-5) * g_ref[...]
    shift = be_ref[...] - mu * scale
    return scale, shift


def _k2(x_ref, s1_ref, q1_ref, g_ref, be_ref,
        W1_ref, pW1_ref, pb1_ref, W2f_ref,
        h1_ref, y2_ref, s2_ref, q2_ref, w2_scr):
    i = pl.program_id(0)
    f32 = jnp.float32
    bf16 = jnp.bfloat16

    @pl.when(i == 0)
    def _():
        w2_scr[...] = W2f_ref[...].astype(bf16)
        s2_ref[...] = jnp.zeros_like(s2_ref)
        q2_ref[...] = jnp.zeros_like(q2_ref)

    scale, shift = _bn_scale_shift(s1_ref, q1_ref, g_ref, be_ref)
    x = x_ref[...]
    y1 = jnp.dot(x, W1_ref[...], preferred_element_type=f32)
    y1p = jnp.dot(x, pW1_ref[...], preferred_element_type=f32) + pb1_ref[...]
    h1 = jnp.maximum(y1 * scale + shift, 0.0) + y1p
    h1b = h1.astype(bf16)
    h1_ref[...] = h1b
    y2 = jnp.dot(h1b, w2_scr[...], preferred_element_type=f32)
    y2b = y2.astype(bf16)
    y2_ref[...] = y2b
    y2f = y2b.astype(f32)

    s2_ref[...] += jnp.sum(y2f, axis=0, keepdims=True)
    q2_ref[...] += jnp.sum(y2f * y2f, axis=0, keepdims=True)


def _k3(h1_ref, y2_ref, s_ref, q_ref, g_ref, be_ref, W3f_ref, pW3f_ref,
        pb3_ref, ow_ref, y3_ref, lo3_ref, s3_ref, q3_ref, w3_scr, pw3_scr):
    i = pl.program_id(0)
    f32 = jnp.float32
    bf16 = jnp.bfloat16

    @pl.when(i == 0)
    def _():
        w3_scr[...] = W3f_ref[...].astype(bf16)
        pw3_scr[...] = pW3f_ref[...].astype(bf16)
        s3_ref[...] = jnp.zeros_like(s3_ref)
        q3_ref[...] = jnp.zeros_like(q3_ref)

    scale, shift = _bn_scale_shift(s_ref, q_ref, g_ref, be_ref)
    h2 = (jnp.maximum(y2_ref[...].astype(f32) * scale + shift, 0.0)
          + h1_ref[...].astype(f32))
    h2b = h2.astype(bf16)
    y3 = jnp.dot(h2b, w3_scr[...], preferred_element_type=f32)
    y3p = jnp.dot(h2b, pw3_scr[...], preferred_element_type=f32) + pb3_ref[...]
    y3b = y3.astype(bf16)
    y3_ref[...] = y3b
    y3f = y3b.astype(f32)
    lo3_ref[...] = jnp.sum(y3p * ow_ref[...], axis=1, keepdims=True)

    s3_ref[...] += jnp.sum(y3f, axis=0, keepdims=True)
    q3_ref[...] += jnp.sum(y3f * y3f, axis=0, keepdims=True)


def _k4(y3_ref, lo3_ref, s_ref, q_ref, g_ref, be_ref, ow_ref, ob_ref, out_ref):
    scale, shift = _bn_scale_shift(s_ref, q_ref, g_ref, be_ref)
    h3r = jnp.maximum(y3_ref[...].astype(jnp.float32) * scale + shift, 0.0)
    out_ref[...] = (jnp.sum(h3r * ow_ref[...], axis=1, keepdims=True)
                    + lo3_ref[...] + ob_ref[...])


def _const_spec(shape):
    nd = len(shape)
    return pl.BlockSpec(shape, lambda i: (0,) * nd)


def _tile_spec(cols, rows=_TILE):
    return pl.BlockSpec((rows, cols), lambda i: (i, 0))


_SEQ = pltpu.CompilerParams(dimension_semantics=("arbitrary",))


def kernel(user_id, gender, age, occupation, movie_id, genres, rating, implicit,
           emb_user, emb_gender, emb_age, emb_occ, emb_movie,
           genre_W, genre_b, ug_W, ug_b,
           b1_W, b1_b, b1_g, b1_beta, b1_pW, b1_pb,
           b2_W, b2_b, b2_g, b2_beta,
           b3_W, b3_b, b3_g, b3_beta, b3_pW, b3_pb,
           out_W, out_b):
    f32 = jnp.float32
    bf16 = jnp.bfloat16
    idx3 = jnp.concatenate([
        user_id.astype(jnp.int32),
        movie_id.astype(jnp.int32),
        (gender * 256 + age * 32 + occupation).astype(jnp.int32),
    ])

    big = jnp.concatenate([emb_user, emb_movie], axis=1)
    g1024 = jnp.arange(1024)
    combo = jnp.concatenate(
        [emb_gender[g1024 // 256], emb_age[(g1024 // 32) % 8],
         emb_occ[g1024 % 32], jnp.zeros((1024, 96), f32)], axis=1)

    u128, m128, c128 = _sc_gather3(idx3, big, combo)

    T = ug_W.reshape(64, 1024).astype(bf16)
    c1024 = jnp.arange(1024)
    Qw = jnp.pad((jnp.arange(32)[:, None] == (c1024[None, :] // 32)
                  ).astype(bf16), ((32, 64), (0, 0)))
    Pw = jnp.pad(((c1024[:, None] % 32) == jnp.arange(32)[None, :]
                  ).astype(bf16), ((0, 0), (64, 32)))
    GWw = jnp.pad(genre_W, ((0, 0), (32, 64)))
    gbw = jnp.pad(genre_b[None, :], ((0, 0), (32, 64)))
    ugbw = jnp.pad(ug_b[None, :], ((0, 0), (64, 32)))

    def permute_w1(W):
        return jnp.concatenate(
            [W[0:64], W[96:160], W[64:96], W[160:192], W[194:226],
             W[192:193], W[192:193], W[193:194],
             jnp.zeros((29, W.shape[1]), f32)], axis=0)

    W1b = permute_w1(b1_W).astype(bf16)
    pW1b = permute_w1(b1_pW).astype(bf16)

    x16, s1, q1 = pl.pallas_call(
        _k1,
        grid=(_GRID,),
        in_specs=[
            _tile_spec(128), _tile_spec(128), _tile_spec(128),
            _tile_spec(19), _tile_spec(1), _tile_spec(1),
            _const_spec((64, 1024)), _const_spec((128, 1024)),
            _const_spec((1024, 128)), _const_spec((19, 128)),
            _const_spec((1, 128)), _const_spec((1, 128)),
            _const_spec((_F, 1024)),
        ],
        out_specs=[
            _tile_spec(_F),
            _const_spec((1, 1024)), _const_spec((1, 1024)),
        ],
        out_shape=[
            jax.ShapeDtypeStruct((_B, _F), bf16),
            jax.ShapeDtypeStruct((1, 1024), f32),
            jax.ShapeDtypeStruct((1, 1024), f32),
        ],
        compiler_params=_SEQ,
    )(u128, m128, c128, genres, rating[:, None], implicit[:, None],
      T, Qw, Pw, GWw, gbw, ugbw, W1b)

    h1, y2, s2, q2 = pl.pallas_call(
        _k2,
        grid=(_GRID,),
        in_specs=[
            _tile_spec(_F),
            _const_spec((1, 1024)), _const_spec((1, 1024)),
            _const_spec((1, 1024)), _const_spec((1, 1024)),
            _const_spec((_F, 1024)), _const_spec((_F, 1024)),
            _const_spec((1, 1024)), _const_spec((1024, 1024)),
        ],
        out_specs=[
            _tile_spec(1024), _tile_spec(1024),
            _const_spec((1, 1024)), _const_spec((1, 1024)),
        ],
        out_shape=[
            jax.ShapeDtypeStruct((_B, 1024), bf16),
            jax.ShapeDtypeStruct((_B, 1024), bf16),
            jax.ShapeDtypeStruct((1, 1024), f32),
            jax.ShapeDtypeStruct((1, 1024), f32),
        ],
        scratch_shapes=[pltpu.VMEM((1024, 1024), bf16)],
        compiler_params=_SEQ,
    )(x16, s1, q1, b1_g.reshape(1, 1024), b1_beta.reshape(1, 1024),
      W1b, pW1b, b1_pb.reshape(1, 1024), b2_W)

    ow = out_W.reshape(1, 512)
    t3 = 1024
    y3, lo3, s3, q3 = pl.pallas_call(
        _k3,
        grid=(_B // t3,),
        in_specs=[
            _tile_spec(1024, t3), _tile_spec(1024, t3),
            _const_spec((1, 1024)), _const_spec((1, 1024)),
            _const_spec((1, 1024)), _const_spec((1, 1024)),
            _const_spec((1024, 512)), _const_spec((1024, 512)),
            _const_spec((1, 512)), _const_spec((1, 512)),
        ],
        out_specs=[
            _tile_spec(512, t3), _tile_spec(1, t3),
            _const_spec((1, 512)), _const_spec((1, 512)),
        ],
        out_shape=[
            jax.ShapeDtypeStruct((_B, 512), bf16),
            jax.ShapeDtypeStruct((_B, 1), f32),
            jax.ShapeDtypeStruct((1, 512), f32),
            jax.ShapeDtypeStruct((1, 512), f32),
        ],
        scratch_shapes=[pltpu.VMEM((1024, 512), bf16),
                        pltpu.VMEM((1024, 512), bf16)],
        compiler_params=_SEQ,
    )(h1, y2, s2, q2, b2_g.reshape(1, 1024), b2_beta.reshape(1, 1024),
      b3_W, b3_pW, b3_pb.reshape(1, 512), ow)

    t4 = 2048
    out = pl.pallas_call(
        _k4,
        grid=(_B // t4,),
        in_specs=[
            _tile_spec(512, t4), _tile_spec(1, t4),
            _const_spec((1, 512)), _const_spec((1, 512)),
            _const_spec((1, 512)), _const_spec((1, 512)),
            _const_spec((1, 512)), _const_spec((1, 1)),
        ],
        out_specs=_tile_spec(1, t4),
        out_shape=jax.ShapeDtypeStruct((_B, 1), f32),
        compiler_params=_SEQ,
    )(y3, lo3, s3, q3, b3_g.reshape(1, 512), b3_beta.reshape(1, 512),
      ow, out_b.reshape(1, 1))

    return out[:, 0]
